# baseline (speedup 1.0000x reference)
import functools

import jax
import jax.numpy as jnp
from jax import lax
from jax.experimental import pallas as pl
from jax.experimental.pallas import tpu as pltpu
from jax.experimental.pallas import tpu_sc as plsc

_N = 10000
_E = 320000
_H = 128

_NCORE = 2
_NSUB = 16
_NTILE = _NCORE * _NSUB
_K = 80
_NBUF = 2
_EPW = 10240
_NCHUNK = _EPW // _K
_NPH = 2
_CPP = _NCHUNK // _NPH
_ROWS_PER_SUB = 632
_NPAD = _ROWS_PER_SUB * _NSUB
_CPAD = 640 * _NSUB
_DUMP = 10000

_mesh = plsc.VectorSubcoreMesh(core_axis_name="c", subcore_axis_name="s")


def _sc_agg_body(with_cnt, *refs):
    if with_cnt:
        (h_hbm, src_hbm, dst_hbm, zrow_hbm, zcnt_hbm,
         agg_out, cnt_out, src_v, dst_v, r0, r1, ones_v,
         agg_sh, cnt_sh, s0, s1) = refs
    else:
        (h_hbm, src_hbm, dst_hbm, zrow_hbm,
         agg_out, src_v, dst_v, r0, r1,
         agg_sh, s0, s1) = refs
    c = lax.axis_index("c")
    s = lax.axis_index("s")
    wid = c * _NSUB + s

    pltpu.sync_copy(zrow_hbm, agg_sh.at[pl.ds(s * _ROWS_PER_SUB, _ROWS_PER_SUB)])
    if with_cnt:
        pltpu.sync_copy(zcnt_hbm, cnt_sh.at[pl.ds(s * 640, 640)])
        for j in range(_K // 16):
            ones_v[pl.ds(j * 16, 16)] = jnp.full((16,), 1.0, jnp.float32)

    plsc.subcore_barrier()

    npair = _CPP // 2

    def scat(b, i):
        dst_row = dst_v.at[i]
        pltpu.sync_copy(b, agg_sh.at[dst_row], add=True)
        if with_cnt:
            pltpu.sync_copy(ones_v, cnt_sh.at[dst_row], add=True)

    for ph in range(_NPH):
        pltpu.sync_copy(src_hbm.at[wid, ph], src_v)
        pltpu.sync_copy(dst_hbm.at[wid, ph], dst_v)
        def chunk(p, carry):
            pltpu.async_copy(h_hbm.at[src_v.at[p]], r0, s0).wait()
            scat(r0, p)
            return carry

        lax.fori_loop(0, _CPP, chunk, 0)
    plsc.subcore_barrier()

    rs = pl.ds(s * _ROWS_PER_SUB, _ROWS_PER_SUB)
    pltpu.sync_copy(agg_sh.at[rs], agg_out.at[c, rs])
    if with_cnt:
        cs = pl.ds(s * 640, 640)
        pltpu.sync_copy(cnt_sh.at[cs], cnt_out.at[c, cs])


def _row_bufs():
    return [pltpu.VMEM((_K, _H), jnp.float32) for _ in range(_NBUF)]


_sc_agg_cnt = functools.partial(
    pl.kernel,
    out_type=(
        jax.ShapeDtypeStruct((_NCORE, _NPAD, _H), jnp.float32),
        jax.ShapeDtypeStruct((_NCORE, _CPAD), jnp.float32),
    ),
    mesh=_mesh,
    scratch_types=[
        pltpu.VMEM((_CPP, _K), jnp.int32),
        pltpu.VMEM((_CPP, _K), jnp.int32),
        *_row_bufs(),
        pltpu.VMEM((_K,), jnp.float32),
        pltpu.VMEM_SHARED((_NPAD, _H), jnp.float32),
        pltpu.VMEM_SHARED((_CPAD,), jnp.float32),
        pltpu.SemaphoreType.DMA,
        pltpu.SemaphoreType.DMA,
    ],
)(functools.partial(_sc_agg_body, True))

_sc_agg = functools.partial(
    pl.kernel,
    out_type=jax.ShapeDtypeStruct((_NCORE, _NPAD, _H), jnp.float32),
    mesh=_mesh,
    scratch_types=[
        pltpu.VMEM((_CPP, _K), jnp.int32),
        pltpu.VMEM((_CPP, _K), jnp.int32),
        *_row_bufs(),
        pltpu.VMEM_SHARED((_NPAD, _H), jnp.float32),
        pltpu.SemaphoreType.DMA,
        pltpu.SemaphoreType.DMA,
    ],
)(functools.partial(_sc_agg_body, False))


_BN = 1000


def _tc_layer_body(p0, p1, c0, c1, h, wl, bl, wr, out):
    cnt = c0[...] + c1[...]
    inv = 1.0 / jnp.maximum(cnt, 1.0)
    mean = (p0[0] + p1[0]) * inv
    acc = jnp.dot(mean, wl[...], preferred_element_type=jnp.float32)
    acc += jnp.dot(h[...], wr[...], preferred_element_type=jnp.float32)
    out[...] = jnp.maximum(acc + bl[...], 0.0)


_p0spec = pl.BlockSpec((1, _BN, _H), lambda i: (0, i, 0))
_p1spec = pl.BlockSpec((1, _BN, _H), lambda i: (1, i, 0))


def _tc_layer(agg, c0, c1, h, wl, bl, wr):
    nb = _N // _BN
    big = pl.BlockSpec((_BN, _H), lambda i: (i, 0))
    col = pl.BlockSpec((_BN, 1), lambda i: (i, 0))
    wspec = pl.BlockSpec((_H, _H), lambda i: (0, 0))
    bspec = pl.BlockSpec((1, _H), lambda i: (0, 0))
    return pl.pallas_call(
        _tc_layer_body,
        grid=(nb,),
        in_specs=[_p0spec, _p1spec, col, col, big, wspec, bspec, wspec],
        out_specs=big,
        out_shape=jax.ShapeDtypeStruct((_N, _H), jnp.float32),
    )(agg, agg, c0, c1, h, wl, bl, wr)


def _tc_layer3_head_body(p0, p1, c0, c1, h, wl, bl, wr, w1, b1, w2, b2, out):
    cnt = c0[...] + c1[...]
    inv = 1.0 / jnp.maximum(cnt, 1.0)
    mean = (p0[0] + p1[0]) * inv
    acc = jnp.dot(mean, wl[...], preferred_element_type=jnp.float32)
    acc += jnp.dot(h[...], wr[...], preferred_element_type=jnp.float32)
    h3 = jnp.maximum(acc + bl[...], 0.0)
    t = jnp.maximum(
        jnp.dot(h3, w1[...], preferred_element_type=jnp.float32) + b1[...], 0.0)
    out[...] = jnp.dot(t, w2[...], preferred_element_type=jnp.float32) + b2[...]


def _tc_layer3_head(agg, c0, c1, h, wl, bl, wr, w1, b1, w2, b2):
    nb = _N // _BN
    big = pl.BlockSpec((_BN, _H), lambda i: (i, 0))
    col = pl.BlockSpec((_BN, 1), lambda i: (i, 0))
    full = lambda a: pl.BlockSpec(a.shape, lambda i: tuple(0 for _ in a.shape))
    return pl.pallas_call(
        _tc_layer3_head_body,
        grid=(nb,),
        in_specs=[_p0spec, _p1spec, col, col, big, full(wl), full(bl), full(wr),
                  full(w1), full(b1), full(w2), full(b2)],
        out_specs=pl.BlockSpec((_BN, 2), lambda i: (i, 0)),
        out_shape=jax.ShapeDtypeStruct((_N, 2), jnp.float32),
    )(agg, agg, c0, c1, h, wl, bl, wr, w1, b1, w2, b2)


def kernel(x, edge_index, Wl0, bl0, Wr0, Wl1, bl1, Wr1, Wl2, bl2, Wr2,
           W1, b1, W2, b2):
    pad = _EPW - _E // _NTILE
    src = edge_index[0].reshape(_NTILE, _E // _NTILE)
    dst = edge_index[1].reshape(_NTILE, _E // _NTILE)
    src = jnp.pad(src, ((0, 0), (0, pad))).reshape(_NTILE, _NPH, _CPP, _K)
    padt = _DUMP + jnp.arange(pad, dtype=jnp.int32) % (_NPAD - _DUMP)
    dst = jnp.concatenate([dst, jnp.broadcast_to(padt, (_NTILE, pad))], axis=1)
    dst = dst.reshape(_NTILE, _NPH, _CPP, _K)
    zrow = jnp.zeros((_ROWS_PER_SUB, _H), jnp.float32)
    zcnt = jnp.zeros((640,), jnp.float32)
    bl0r, bl1r, bl2r = (b.reshape(1, -1) for b in (bl0, bl1, bl2))
    b1r = b1.reshape(1, -1)
    b2r = b2.reshape(1, -1)

    agg, cnt = _sc_agg_cnt(x, src, dst, zrow, zcnt)
    c0 = cnt[0, :_N].reshape(_N, 1)
    c1 = cnt[1, :_N].reshape(_N, 1)
    h1 = _tc_layer(agg, c0, c1, x, Wl0, bl0r, Wr0)
    agg = _sc_agg(h1, src, dst, zrow)
    h2 = _tc_layer(agg, c0, c1, h1, Wl1, bl1r, Wr1)
    agg = _sc_agg(h2, src, dst, zrow)
    return _tc_layer3_head(agg, c0, c1, h2, Wl2, bl2r, Wr2,
                           W1, b1r, W2, b2r)


# speedup vs baseline: 3.1971x; 3.1971x over previous
import functools

import jax
import jax.numpy as jnp
from jax import lax
from jax.experimental import pallas as pl
from jax.experimental.pallas import tpu as pltpu
from jax.experimental.pallas import tpu_sc as plsc

_N = 10000
_E = 320000
_H = 128
_HH = 64

_NCORE = 2
_NSUB = 16
_K = 160
_EPW = _E // _NSUB
_NCHUNK = _EPW // _K
_ROWS_PER_SUB = 632
_NPAD = _ROWS_PER_SUB * _NSUB
_CPAD = 640 * _NSUB

_mesh = plsc.VectorSubcoreMesh(core_axis_name="c", subcore_axis_name="s")


def _sc_agg_body(with_cnt, *refs):
    if with_cnt:
        (h_hbm, pk_hbm, zrow_hbm, zcnt_hbm,
         agg_out, cnt_out, pk_v, sb0, db0, sb1, db1, r0, r1, ones_v,
         agg_sh, cnt_sh, s0, s1) = refs
    else:
        (h_hbm, pk_hbm, zrow_hbm,
         agg_out, pk_v, sb0, db0, sb1, db1, r0, r1,
         agg_sh, s0, s1) = refs
    c = lax.axis_index("c")
    s = lax.axis_index("s")
    half = c * _N

    pltpu.sync_copy(zrow_hbm, agg_sh.at[pl.ds(s * _ROWS_PER_SUB, _ROWS_PER_SUB)])
    if with_cnt:
        pltpu.sync_copy(zcnt_hbm, cnt_sh.at[pl.ds(s * 640, 640)])
        for j in range(_K // 16):
            ones_v[pl.ds(j * 16, 16)] = jnp.full((16,), 1.0, jnp.float32)

    pltpu.sync_copy(pk_hbm.at[s], pk_v)
    plsc.subcore_barrier()

    def unpack(ch, sb, db):
        for j in range(_K // 16):
            w = pk_v[ch, pl.ds(j * 16, 16)]
            sb[pl.ds(j * 16, 16)] = (w & 0xFFFF) + half
            db[pl.ds(j * 16, 16)] = w >> 16

    def scat(b, db):
        pltpu.sync_copy(b, agg_sh.at[db], add=True)
        if with_cnt:
            pltpu.sync_copy(ones_v, cnt_sh.at[db], add=True)

    unpack(0, sb0, db0)
    pltpu.async_copy(h_hbm.at[sb0], r0, s0)

    def pair(p, carry):
        unpack(2 * p + 1, sb1, db1)
        pltpu.async_copy(h_hbm.at[sb1], r1, s1)
        pltpu.make_async_copy(h_hbm.at[sb0], r0, s0).wait()
        scat(r0, db0)
        unpack(2 * p + 2, sb0, db0)
        pltpu.async_copy(h_hbm.at[sb0], r0, s0)
        pltpu.make_async_copy(h_hbm.at[sb1], r1, s1).wait()
        scat(r1, db1)
        return carry

    lax.fori_loop(0, (_NCHUNK - 1) // 2, pair, 0)
    pltpu.make_async_copy(h_hbm.at[sb0], r0, s0).wait()
    scat(r0, db0)
    plsc.subcore_barrier()

    rs = pl.ds(s * _ROWS_PER_SUB, _ROWS_PER_SUB)
    pltpu.sync_copy(agg_sh.at[rs], agg_out.at[c, rs])
    if with_cnt:
        cs = pl.ds(s * 640, 640)
        pltpu.sync_copy(cnt_sh.at[cs], cnt_out.at[c, cs])


def _sc_scratch():
    return [
        pltpu.VMEM((_NCHUNK, _K), jnp.int32),
        pltpu.VMEM((_K,), jnp.int32),
        pltpu.VMEM((_K,), jnp.int32),
        pltpu.VMEM((_K,), jnp.int32),
        pltpu.VMEM((_K,), jnp.int32),
        pltpu.VMEM((_K, _HH), jnp.float32),
        pltpu.VMEM((_K, _HH), jnp.float32),
    ]


_sc_agg_cnt = functools.partial(
    pl.kernel,
    compiler_params=pltpu.CompilerParams(use_tc_tiling_on_sc=False),
    out_type=(
        jax.ShapeDtypeStruct((_NCORE, _NPAD, _HH), jnp.float32),
        jax.ShapeDtypeStruct((_NCORE, _CPAD), jnp.float32),
    ),
    mesh=_mesh,
    scratch_types=[
        *_sc_scratch(),
        pltpu.VMEM((_K,), jnp.float32),
        pltpu.VMEM_SHARED((_NPAD, _HH), jnp.float32),
        pltpu.VMEM_SHARED((_CPAD,), jnp.float32),
        pltpu.SemaphoreType.DMA,
        pltpu.SemaphoreType.DMA,
    ],
)(functools.partial(_sc_agg_body, True))

_sc_agg = functools.partial(
    pl.kernel,
    compiler_params=pltpu.CompilerParams(use_tc_tiling_on_sc=False),
    out_type=jax.ShapeDtypeStruct((_NCORE, _NPAD, _HH), jnp.float32),
    mesh=_mesh,
    scratch_types=[
        *_sc_scratch(),
        pltpu.VMEM_SHARED((_NPAD, _HH), jnp.float32),
        pltpu.SemaphoreType.DMA,
        pltpu.SemaphoreType.DMA,
    ],
)(functools.partial(_sc_agg_body, False))


_BN = 1000

_p0spec = pl.BlockSpec((1, _BN, _HH), lambda i: (0, i, 0))
_p1spec = pl.BlockSpec((1, _BN, _HH), lambda i: (1, i, 0))
_hsplit_out = pl.BlockSpec((2, _BN, _HH), lambda i: (0, i, 0))


def _mean_plus_lin(p0, p1, c0, h0, h1, wl, bl, wr):
    inv = 1.0 / jnp.maximum(c0[...], 1.0)
    mean = jnp.concatenate([p0[0], p1[0]], axis=1) * inv
    h = jnp.concatenate([h0[0], h1[0]], axis=1)
    acc = jnp.dot(mean, wl[...], preferred_element_type=jnp.float32)
    acc += jnp.dot(h, wr[...], preferred_element_type=jnp.float32)
    return jnp.maximum(acc + bl[...], 0.0)


def _tc_layer_body(p0, p1, c0, h0, h1, wl, bl, wr, out):
    acts = _mean_plus_lin(p0, p1, c0, h0, h1, wl, bl, wr)
    out[0] = acts[:, :_HH]
    out[1] = acts[:, _HH:]


def _tc_layer(agg, c0, hs, wl, bl, wr):
    nb = _N // _BN
    col = pl.BlockSpec((_BN, 1), lambda i: (i, 0))
    wspec = pl.BlockSpec((_H, _H), lambda i: (0, 0))
    bspec = pl.BlockSpec((1, _H), lambda i: (0, 0))
    return pl.pallas_call(
        _tc_layer_body,
        grid=(nb,),
        in_specs=[_p0spec, _p1spec, col, _p0spec, _p1spec,
                  wspec, bspec, wspec],
        out_specs=_hsplit_out,
        out_shape=jax.ShapeDtypeStruct((2, _N, _HH), jnp.float32),
    )(agg, agg, c0, hs, hs, wl, bl, wr)


def _tc_layer3_head_body(p0, p1, c0, h0, h1, wl, bl, wr, w1, b1, w2, b2, out):
    h3 = _mean_plus_lin(p0, p1, c0, h0, h1, wl, bl, wr)
    t = jnp.maximum(
        jnp.dot(h3, w1[...], preferred_element_type=jnp.float32) + b1[...], 0.0)
    out[...] = jnp.dot(t, w2[...], preferred_element_type=jnp.float32) + b2[...]


def _tc_layer3_head(agg, c0, hs, wl, bl, wr, w1, b1, w2, b2):
    nb = _N // _BN
    col = pl.BlockSpec((_BN, 1), lambda i: (i, 0))
    full = lambda a: pl.BlockSpec(a.shape, lambda i: tuple(0 for _ in a.shape))
    return pl.pallas_call(
        _tc_layer3_head_body,
        grid=(nb,),
        in_specs=[_p0spec, _p1spec, col, _p0spec, _p1spec,
                  full(wl), full(bl), full(wr),
                  full(w1), full(b1), full(w2), full(b2)],
        out_specs=pl.BlockSpec((_BN, 2), lambda i: (i, 0)),
        out_shape=jax.ShapeDtypeStruct((_N, 2), jnp.float32),
    )(agg, agg, c0, hs, hs, wl, bl, wr, w1, b1, w2, b2)


def kernel(x, edge_index, Wl0, bl0, Wr0, Wl1, bl1, Wr1, Wl2, bl2, Wr2,
           W1, b1, W2, b2):
    pk = (edge_index[1] << 16) | edge_index[0]
    pk = pk.reshape(_NSUB, _NCHUNK, _K)
    zrow = jnp.zeros((_ROWS_PER_SUB, _HH), jnp.float32)
    zcnt = jnp.zeros((640,), jnp.float32)
    bl0r, bl1r, bl2r = (b.reshape(1, -1) for b in (bl0, bl1, bl2))
    b1r = b1.reshape(1, -1)
    b2r = b2.reshape(1, -1)

    xs = jnp.stack([x[:, :_HH], x[:, _HH:]])
    agg, cnt = _sc_agg_cnt(xs.reshape(2 * _N, _HH), pk, zrow, zcnt)
    c0 = cnt[0, :_N].reshape(_N, 1)
    h = _tc_layer(agg, c0, xs, Wl0, bl0r, Wr0)
    agg = _sc_agg(h.reshape(2 * _N, _HH), pk, zrow)
    h = _tc_layer(agg, c0, h, Wl1, bl1r, Wr1)
    agg = _sc_agg(h.reshape(2 * _N, _HH), pk, zrow)
    return _tc_layer3_head(agg, c0, h, Wl2, bl2r, Wr2, W1, b1r, W2, b2r)


# speedup vs baseline: 3.4726x; 1.0862x over previous
import functools

import jax
import jax.numpy as jnp
from jax import lax
from jax.experimental import pallas as pl
from jax.experimental.pallas import tpu as pltpu
from jax.experimental.pallas import tpu_sc as plsc

_N = 10000
_E = 320000
_H = 128

_NCORE = 2
_NSUB = 16
_NTILE = _NCORE * _NSUB
_K = 80
_EPW = _E // _NTILE
_NCHUNK = _EPW // _K
_ROWS_PER_SUB = 632
_NPAD = _ROWS_PER_SUB * _NSUB
_CPAD = 640 * _NSUB

_mesh = plsc.VectorSubcoreMesh(core_axis_name="c", subcore_axis_name="s")


def _sc_agg_body(with_cnt, *refs):
    if with_cnt:
        (h_hbm, pk_hbm, zrow_hbm, zcnt_hbm,
         agg_out, cnt_out, pk_v, sb0, db0, sb1, db1, r0, r1, ones_v,
         agg_sh, cnt_sh, s0, s1) = refs
    else:
        (h_hbm, pk_hbm, zrow_hbm,
         agg_out, pk_v, sb0, db0, sb1, db1, r0, r1,
         agg_sh, s0, s1) = refs
    c = lax.axis_index("c")
    s = lax.axis_index("s")
    wid = c * _NSUB + s

    pltpu.sync_copy(zrow_hbm, agg_sh.at[pl.ds(s * _ROWS_PER_SUB, _ROWS_PER_SUB)])
    if with_cnt:
        pltpu.sync_copy(zcnt_hbm, cnt_sh.at[pl.ds(s * 640, 640)])
        for j in range(_K // 16):
            ones_v[pl.ds(j * 16, 16)] = jnp.full((16,), 1.0, jnp.float32)

    pltpu.sync_copy(pk_hbm.at[wid], pk_v)
    plsc.subcore_barrier()

    def unpack(ch, sb, db):
        for j in range(_K // 16):
            w = pk_v[ch, pl.ds(j * 16, 16)]
            sb[pl.ds(j * 16, 16)] = w & 0xFFFF
            db[pl.ds(j * 16, 16)] = w >> 16

    def scat(b, db):
        pltpu.sync_copy(b, agg_sh.at[db], add=True)
        if with_cnt:
            pltpu.sync_copy(ones_v, cnt_sh.at[db], add=True)

    unpack(0, sb0, db0)
    pltpu.async_copy(h_hbm.at[sb0], r0, s0)

    def pair(p, carry):
        unpack(2 * p + 1, sb1, db1)
        pltpu.async_copy(h_hbm.at[sb1], r1, s1)
        pltpu.make_async_copy(h_hbm.at[sb0], r0, s0).wait()
        scat(r0, db0)
        unpack(2 * p + 2, sb0, db0)
        pltpu.async_copy(h_hbm.at[sb0], r0, s0)
        pltpu.make_async_copy(h_hbm.at[sb1], r1, s1).wait()
        scat(r1, db1)
        return carry

    lax.fori_loop(0, (_NCHUNK - 1) // 2, pair, 0)
    pltpu.make_async_copy(h_hbm.at[sb0], r0, s0).wait()
    scat(r0, db0)
    plsc.subcore_barrier()

    rs = pl.ds(s * _ROWS_PER_SUB, _ROWS_PER_SUB)
    pltpu.sync_copy(agg_sh.at[rs], agg_out.at[c, rs])
    if with_cnt:
        cs = pl.ds(s * 640, 640)
        pltpu.sync_copy(cnt_sh.at[cs], cnt_out.at[c, cs])


def _row_bufs():
    return [pltpu.VMEM((_K, _H), jnp.float32),
            pltpu.VMEM((_K, _H), jnp.float32)]


_sc_agg_cnt = functools.partial(
    pl.kernel,
    out_type=(
        jax.ShapeDtypeStruct((_NCORE, _NPAD, _H), jnp.float32),
        jax.ShapeDtypeStruct((_NCORE, _CPAD), jnp.float32),
    ),
    mesh=_mesh,
    scratch_types=[
        pltpu.VMEM((_NCHUNK, _K), jnp.int32),
        pltpu.VMEM((_K,), jnp.int32),
        pltpu.VMEM((_K,), jnp.int32),
        pltpu.VMEM((_K,), jnp.int32),
        pltpu.VMEM((_K,), jnp.int32),
        *_row_bufs(),
        pltpu.VMEM((_K,), jnp.float32),
        pltpu.VMEM_SHARED((_NPAD, _H), jnp.float32),
        pltpu.VMEM_SHARED((_CPAD,), jnp.float32),
        pltpu.SemaphoreType.DMA,
        pltpu.SemaphoreType.DMA,
    ],
)(functools.partial(_sc_agg_body, True))

_sc_agg = functools.partial(
    pl.kernel,
    out_type=jax.ShapeDtypeStruct((_NCORE, _NPAD, _H), jnp.float32),
    mesh=_mesh,
    scratch_types=[
        pltpu.VMEM((_NCHUNK, _K), jnp.int32),
        pltpu.VMEM((_K,), jnp.int32),
        pltpu.VMEM((_K,), jnp.int32),
        pltpu.VMEM((_K,), jnp.int32),
        pltpu.VMEM((_K,), jnp.int32),
        *_row_bufs(),
        pltpu.VMEM_SHARED((_NPAD, _H), jnp.float32),
        pltpu.SemaphoreType.DMA,
        pltpu.SemaphoreType.DMA,
    ],
)(functools.partial(_sc_agg_body, False))


_BN = 1000


def _tc_layer_body(p0, p1, c0, c1, h, wl, bl, wr, out):
    cnt = c0[...] + c1[...]
    inv = 1.0 / jnp.maximum(cnt, 1.0)
    mean = (p0[0] + p1[0]) * inv
    acc = jnp.dot(mean, wl[...], preferred_element_type=jnp.float32)
    acc += jnp.dot(h[...], wr[...], preferred_element_type=jnp.float32)
    out[...] = jnp.maximum(acc + bl[...], 0.0)


_p0spec = pl.BlockSpec((1, _BN, _H), lambda i: (0, i, 0))
_p1spec = pl.BlockSpec((1, _BN, _H), lambda i: (1, i, 0))


def _tc_layer(agg, c0, c1, h, wl, bl, wr):
    nb = _N // _BN
    big = pl.BlockSpec((_BN, _H), lambda i: (i, 0))
    col = pl.BlockSpec((_BN, 1), lambda i: (i, 0))
    wspec = pl.BlockSpec((_H, _H), lambda i: (0, 0))
    bspec = pl.BlockSpec((1, _H), lambda i: (0, 0))
    return pl.pallas_call(
        _tc_layer_body,
        grid=(nb,),
        in_specs=[_p0spec, _p1spec, col, col, big, wspec, bspec, wspec],
        out_specs=big,
        out_shape=jax.ShapeDtypeStruct((_N, _H), jnp.float32),
    )(agg, agg, c0, c1, h, wl, bl, wr)


def _tc_layer3_head_body(p0, p1, c0, c1, h, wl, bl, wr, w1, b1, w2, b2, out):
    cnt = c0[...] + c1[...]
    inv = 1.0 / jnp.maximum(cnt, 1.0)
    mean = (p0[0] + p1[0]) * inv
    acc = jnp.dot(mean, wl[...], preferred_element_type=jnp.float32)
    acc += jnp.dot(h[...], wr[...], preferred_element_type=jnp.float32)
    h3 = jnp.maximum(acc + bl[...], 0.0)
    t = jnp.maximum(
        jnp.dot(h3, w1[...], preferred_element_type=jnp.float32) + b1[...], 0.0)
    out[...] = jnp.dot(t, w2[...], preferred_element_type=jnp.float32) + b2[...]


def _tc_layer3_head(agg, c0, c1, h, wl, bl, wr, w1, b1, w2, b2):
    nb = _N // _BN
    big = pl.BlockSpec((_BN, _H), lambda i: (i, 0))
    col = pl.BlockSpec((_BN, 1), lambda i: (i, 0))
    full = lambda a: pl.BlockSpec(a.shape, lambda i: tuple(0 for _ in a.shape))
    return pl.pallas_call(
        _tc_layer3_head_body,
        grid=(nb,),
        in_specs=[_p0spec, _p1spec, col, col, big, full(wl), full(bl), full(wr),
                  full(w1), full(b1), full(w2), full(b2)],
        out_specs=pl.BlockSpec((_BN, 2), lambda i: (i, 0)),
        out_shape=jax.ShapeDtypeStruct((_N, 2), jnp.float32),
    )(agg, agg, c0, c1, h, wl, bl, wr, w1, b1, w2, b2)


def kernel(x, edge_index, Wl0, bl0, Wr0, Wl1, bl1, Wr1, Wl2, bl2, Wr2,
           W1, b1, W2, b2):
    pk = (edge_index[1] << 16) | edge_index[0]
    pk = pk.reshape(_NTILE, _NCHUNK, _K)
    zrow = jnp.zeros((_ROWS_PER_SUB, _H), jnp.float32)
    zcnt = jnp.zeros((640,), jnp.float32)
    bl0r, bl1r, bl2r = (b.reshape(1, -1) for b in (bl0, bl1, bl2))
    b1r = b1.reshape(1, -1)
    b2r = b2.reshape(1, -1)

    agg, cnt = _sc_agg_cnt(x, pk, zrow, zcnt)
    c0 = cnt[0, :_N].reshape(_N, 1)
    c1 = cnt[1, :_N].reshape(_N, 1)
    h1 = _tc_layer(agg, c0, c1, x, Wl0, bl0r, Wr0)
    agg = _sc_agg(h1, pk, zrow)
    h2 = _tc_layer(agg, c0, c1, h1, Wl1, bl1r, Wr1)
    agg = _sc_agg(h2, pk, zrow)
    return _tc_layer3_head(agg, c0, c1, h2, Wl2, bl2r, Wr2,
                           W1, b1r, W2, b2r)
